# baseline (speedup 1.0000x reference)
import functools
import jax
import jax.numpy as jnp
from jax import lax
from jax.experimental import pallas as pl
from jax.experimental.pallas import tpu as pltpu
from jax.experimental.pallas import tpu_sc as plsc

E = 16
D = 2048
T = 8
NKD = D // 16


def _take(v, idx):
    dn = lax.GatherDimensionNumbers(
        offset_dims=(), collapsed_slice_dims=(0,), start_index_map=(0,))
    return lax.gather(v, idx[:, None], dn, (1,),
                      mode=lax.GatherScatterMode.PROMISE_IN_BOUNDS)


def _bsum(v):
    lane = lax.broadcasted_iota(jnp.int32, (16,), 0)
    for sh in (1, 2, 4, 8):
        v = v + _take(v, lane ^ sh)
    return v


def _bmax(v):
    lane = lax.broadcasted_iota(jnp.int32, (16,), 0)
    for sh in (1, 2, 4, 8):
        v = jnp.maximum(v, _take(v, lane ^ sh))
    return v


def _sc_routing(x_hbm, rw_hbm, fullw_hbm, ids_hbm,
                xbuf, wbuf, fwrow, fwbuf, idsbuf, idsout):
    c = lax.axis_index("c")
    s = lax.axis_index("s")
    lane = lax.broadcasted_iota(jnp.int32, (16,), 0)

    @pl.when(jnp.logical_and(c == 0, s < T))
    def _token_tile():
        t = s
        pltpu.sync_copy(x_hbm.at[t], xbuf)
        pltpu.sync_copy(rw_hbm, wbuf)
        logits = jnp.zeros((16,), jnp.float32)
        for r in range(E):
            def inner(k, acc):
                xa = xbuf[pl.ds(k * 16, 16)]
                wa = wbuf[r, pl.ds(k * 16, 16)]
                return acc + xa * wa
            acc = lax.fori_loop(0, NKD, inner, jnp.zeros((16,), jnp.float32))
            logits = jnp.where(lane == r, _bsum(acc), logits)
        m = _bmax(logits)
        exl = jnp.exp(logits - m)
        p = exl / _bsum(exl)
        p1 = _bmax(p)
        i1 = 16 - _bmax(jnp.where(p == p1, 16 - lane, 0))
        oh1 = lane == i1
        pm = jnp.where(oh1, -1.0, p)
        p2 = _bmax(pm)
        i2 = 16 - _bmax(jnp.where(pm == p2, 16 - lane, 0))
        oh2 = lane == i2
        den = p1 + p2
        row = (jnp.where(oh1, p1 / den, 0.0)
               + jnp.where(oh2, p2 / den, 0.0))
        fwrow[...] = row
        pltpu.sync_copy(fwrow, fullw_hbm.at[t])

    plsc.subcore_barrier()

    @pl.when(jnp.logical_and(c == 0, s == 0))
    def _compact_tile():
        pltpu.sync_copy(fullw_hbm, fwbuf)
        colsum = jnp.zeros((16,), jnp.float32)
        for t in range(T):
            colsum = colsum + fwbuf[t]
        active = colsum > 0.0
        act_i = jnp.where(active, 1, 0)
        nact = _bsum(act_i)
        incl = act_i
        for sh in (1, 2, 4, 8):
            shifted = _take(incl, jnp.maximum(lane - sh, 0))
            incl = incl + jnp.where(lane >= sh, shifted, 0)
        pos = incl - act_i
        lastid = _bmax(jnp.where(active, lane, 0))
        pos_adj = jnp.where(active, pos, 16 + lane)
        idsbuf[pl.ds(0, 16)] = lastid
        idsbuf[pl.ds(16, 16)] = lastid
        plsc.store_scatter(idsbuf, [pos_adj], lane)
        eids_sorted = idsbuf[pl.ds(0, 16)]
        eids = jnp.where(lane < nact, eids_sorted, lastid)
        idsout[...] = eids
        pltpu.sync_copy(idsout, ids_hbm.at[pl.ds(0, 16)])
        idsout[...] = jnp.where(lane == 0, nact, 0)
        pltpu.sync_copy(idsout, ids_hbm.at[pl.ds(16, 16)])


def _expert_kernel(eids_ref, nact_ref, x_ref, fullw_ref,
                   gate_ref, up_ref, down_ref, out_ref):
    i = pl.program_id(0)

    @pl.when(i == 0)
    def _init():
        out_ref[...] = jnp.zeros_like(out_ref)

    @pl.when(i < nact_ref[0])
    def _compute():
        x = x_ref[...]
        g = jax.lax.dot_general(
            x, gate_ref[0], (((1,), (1,)), ((), ())),
            preferred_element_type=jnp.float32)
        u = jax.lax.dot_general(
            x, up_ref[0], (((1,), (1,)), ((), ())),
            preferred_element_type=jnp.float32)
        h = (g * jax.nn.sigmoid(g)) * u
        o = jax.lax.dot_general(
            h, down_ref[0], (((1,), (1,)), ((), ())),
            preferred_element_type=jnp.float32)
        e = eids_ref[i]
        tt, ee = fullw_ref.shape
        elane = jax.lax.broadcasted_iota(jnp.int32, (tt, ee), 1)
        w = jnp.sum(jnp.where(elane == e, fullw_ref[...], 0.0),
                    axis=1, keepdims=True)
        out_ref[...] += w * o


def kernel(hidden_states, router_w, gate_w, up_w, down_w):
    B, S, H = hidden_states.shape
    F = gate_w.shape[1]
    x = hidden_states.reshape(T, D)

    sc = functools.partial(
        pl.kernel,
        out_type=(
            jax.ShapeDtypeStruct((T, E), jnp.float32),
            jax.ShapeDtypeStruct((2 * E,), jnp.int32),
        ),
        mesh=plsc.VectorSubcoreMesh(core_axis_name="c", subcore_axis_name="s"),
        scratch_types=[
            pltpu.VMEM((D,), jnp.float32),
            pltpu.VMEM((E, D), jnp.float32),
            pltpu.VMEM((E,), jnp.float32),
            pltpu.VMEM((T, E), jnp.float32),
            pltpu.VMEM((2 * E,), jnp.int32),
            pltpu.VMEM((E,), jnp.int32),
        ],
        compiler_params=pltpu.CompilerParams(needs_layout_passes=False),
    )(_sc_routing)
    fullw, ids = sc(x, router_w)
    eids = ids[:E]
    nact = ids[E:E + 1]

    out = pl.pallas_call(
        _expert_kernel,
        grid_spec=pltpu.PrefetchScalarGridSpec(
            num_scalar_prefetch=2,
            grid=(E,),
            in_specs=[
                pl.BlockSpec((T, D), lambda i, eids, nact: (0, 0)),
                pl.BlockSpec((T, E), lambda i, eids, nact: (0, 0)),
                pl.BlockSpec((1, F, D), lambda i, eids, nact: (eids[i], 0, 0)),
                pl.BlockSpec((1, F, D), lambda i, eids, nact: (eids[i], 0, 0)),
                pl.BlockSpec((1, D, F), lambda i, eids, nact: (eids[i], 0, 0)),
            ],
            out_specs=pl.BlockSpec((T, D), lambda i, eids, nact: (0, 0)),
        ),
        out_shape=jax.ShapeDtypeStruct((T, D), jnp.float32),
        compiler_params=pltpu.CompilerParams(
            dimension_semantics=("arbitrary",),
        ),
    )(eids, nact, x, fullw, gate_w, up_w, down_w)

    return out.reshape(B, S, H)


# speedup vs baseline: 1.3813x; 1.3813x over previous
import jax
import jax.numpy as jnp
from jax.experimental import pallas as pl
from jax.experimental.pallas import tpu as pltpu


def _moe_kernel(x_ref, rw_ref, gate_hbm, up_hbm, down_hbm, out_ref,
                ids_vmem, ids_smem, gbuf, ubuf, dbuf, wsem, isem):
    T, D = x_ref.shape
    E = rw_ref.shape[0]

    x = x_ref[...]
    logits = jax.lax.dot_general(
        x, rw_ref[...], (((1,), (1,)), ((), ())),
        preferred_element_type=jnp.float32)
    m = jnp.max(logits, axis=1, keepdims=True)
    ex = jnp.exp(logits - m)
    probs = ex / jnp.sum(ex, axis=1, keepdims=True)

    lane = jax.lax.broadcasted_iota(jnp.int32, (T, E), 1)
    p1 = jnp.max(probs, axis=1, keepdims=True)
    i1 = jnp.min(jnp.where(probs == p1, lane, E), axis=1, keepdims=True)
    oh1 = lane == i1
    probs2 = jnp.where(oh1, -1.0, probs)
    p2 = jnp.max(probs2, axis=1, keepdims=True)
    i2 = jnp.min(jnp.where(probs2 == p2, lane, E), axis=1, keepdims=True)
    oh2 = lane == i2
    denom = p1 + p2
    full_w = (jnp.where(oh1, p1 / denom, 0.0)
              + jnp.where(oh2, p2 / denom, 0.0))

    ident = (jax.lax.broadcasted_iota(jnp.int32, (E, E), 0)
             == jax.lax.broadcasted_iota(jnp.int32, (E, E), 1)).astype(jnp.float32)
    tri = (jax.lax.broadcasted_iota(jnp.int32, (E, E), 0)
           <= jax.lax.broadcasted_iota(jnp.int32, (E, E), 1)).astype(jnp.float32)

    def tcol(v_row):
        return jax.lax.dot_general(
            ident, v_row, (((1,), (1,)), ((), ())),
            preferred_element_type=jnp.float32)

    active = (jnp.sum(full_w, axis=0, keepdims=True) > 0.0).astype(jnp.float32)
    cums = jax.lax.dot_general(
        active, tri, (((1,), (0,)), ((), ())),
        preferred_element_type=jnp.float32)
    nact = jnp.sum(active, axis=1, keepdims=True)

    active_col = tcol(active)
    pos_col = tcol(cums) - 1.0
    slot_row = jax.lax.broadcasted_iota(jnp.int32, (1, E), 1).astype(jnp.float32)
    M = active_col * (pos_col == slot_row).astype(jnp.float32)

    e_row = jax.lax.broadcasted_iota(jnp.int32, (1, E), 1).astype(jnp.float32)
    eids = jax.lax.dot_general(
        e_row, M, (((1,), (0,)), ((), ())), preferred_element_type=jnp.float32)
    wsel = jax.lax.dot_general(
        full_w, M, (((1,), (0,)), ((), ())),
        preferred_element_type=jnp.float32)

    ids_vmem[...] = jnp.concatenate(
        [eids.astype(jnp.int32), nact.astype(jnp.int32),
         jnp.zeros((1, E - 1), jnp.int32)], axis=1)
    idcopy = pltpu.make_async_copy(ids_vmem, ids_smem, isem)
    idcopy.start()
    idcopy.wait()

    n = ids_smem[0, E]

    def start_copies(s, slot):
        e = ids_smem[0, s]
        pltpu.make_async_copy(gate_hbm.at[e], gbuf.at[slot], wsem.at[0, slot]).start()
        pltpu.make_async_copy(up_hbm.at[e], ubuf.at[slot], wsem.at[1, slot]).start()
        pltpu.make_async_copy(down_hbm.at[e], dbuf.at[slot], wsem.at[2, slot]).start()


    out_ref[...] = jnp.zeros_like(out_ref)
    start_copies(0, 0)

    @pl.when(n > 1)
    def _warm():
        start_copies(1, 1)

    slot_lane = jax.lax.broadcasted_iota(jnp.int32, (T, E), 1)

    def body(s, carry):
        slot = jax.lax.rem(s, 3)

        @pl.when(s + 2 < n)
        def _prefetch():
            start_copies(s + 2, jax.lax.rem(s + 2, 3))

        pltpu.make_async_copy(gate_hbm.at[0], gbuf.at[slot], wsem.at[0, slot]).wait()
        g = jax.lax.dot_general(
            x, gbuf[slot], (((1,), (1,)), ((), ())),
            preferred_element_type=jnp.float32)
        pltpu.make_async_copy(up_hbm.at[0], ubuf.at[slot], wsem.at[1, slot]).wait()
        u = jax.lax.dot_general(
            x, ubuf[slot], (((1,), (1,)), ((), ())),
            preferred_element_type=jnp.float32)
        h = (g * jax.nn.sigmoid(g)) * u
        pltpu.make_async_copy(down_hbm.at[0], dbuf.at[slot], wsem.at[2, slot]).wait()
        o = jax.lax.dot_general(
            h, dbuf[slot], (((1,), (1,)), ((), ())),
            preferred_element_type=jnp.float32)
        w = jnp.sum(jnp.where(slot_lane == s, wsel, 0.0),
                    axis=1, keepdims=True)
        out_ref[...] += w * o
        return carry

    jax.lax.fori_loop(0, n, body, 0)


def kernel(hidden_states, router_w, gate_w, up_w, down_w):
    B, S, D = hidden_states.shape
    T = B * S
    E = router_w.shape[0]
    F = gate_w.shape[1]
    x = hidden_states.reshape(T, D)

    out = pl.pallas_call(
        _moe_kernel,
        in_specs=[
            pl.BlockSpec((T, D), lambda: (0, 0)),
            pl.BlockSpec((E, D), lambda: (0, 0)),
            pl.BlockSpec(memory_space=pl.MemorySpace.ANY),
            pl.BlockSpec(memory_space=pl.MemorySpace.ANY),
            pl.BlockSpec(memory_space=pl.MemorySpace.ANY),
        ],
        out_specs=pl.BlockSpec((T, D), lambda: (0, 0)),
        out_shape=jax.ShapeDtypeStruct((T, D), jnp.float32),
        compiler_params=pltpu.CompilerParams(
            vmem_limit_bytes=100 * 1024 * 1024,
        ),
        scratch_shapes=[
            pltpu.VMEM((1, 2 * E), jnp.int32),
            pltpu.SMEM((1, 2 * E), jnp.int32),
            pltpu.VMEM((3, F, D), jnp.float32),
            pltpu.VMEM((3, F, D), jnp.float32),
            pltpu.VMEM((3, D, F), jnp.float32),
            pltpu.SemaphoreType.DMA((3, 3)),
            pltpu.SemaphoreType.DMA,
        ],
    )(x, router_w, gate_w, up_w, down_w)

    return out.reshape(B, S, D)
